# baseline (speedup 1.0000x reference)
import jax
import jax.numpy as jnp
from jax import lax
from jax.experimental import pallas as pl
from jax.experimental.pallas import tpu as pltpu
from jax.experimental.pallas import tpu_sc as plsc

N = 10000
NP = 10112
E = 320000
D = 128
NC_PAD = 64
NCORES = 2
NSUB = 16
NW = NCORES * NSUB
ROWS_PER_TILE = NP // NSUB


def _make_sc_agg(dv, with_deg, k, nbuf):
    mesh = plsc.VectorSubcoreMesh(core_axis_name="c", subcore_axis_name="s")

    nchunk = E // k

    out_type = [jax.ShapeDtypeStruct((NCORES, NP, dv), jnp.float32)]
    scratch = [
        pltpu.VMEM((nbuf, k), jnp.int32),
        pltpu.VMEM((nbuf, k), jnp.int32),
        pltpu.VMEM((nbuf, k, dv), jnp.float32),
        pltpu.VMEM_SHARED((NP, dv), jnp.float32),
        pltpu.SemaphoreType.DMA((nbuf,)),
        pltpu.SemaphoreType.DMA((nbuf,)),
        pltpu.SemaphoreType.DMA((nbuf,)),
    ]
    if with_deg:
        out_type.append(jax.ShapeDtypeStruct((NCORES, NP, 16), jnp.float32))
        scratch += [
            pltpu.VMEM((k, 16), jnp.float32),
            pltpu.VMEM_SHARED((NP, 16), jnp.float32),
        ]

    def body(*refs):
        if with_deg:
            (y_hbm, src_hbm, dst_hbm, zeros_hbm, zeros16_hbm, ones_hbm,
             out_hbm, deg_hbm, src_r, dst_r, rows_v, acc_sh, gsem, isem, ssem,
             ones_v, deg_sh) = refs
        else:
            (y_hbm, src_hbm, dst_hbm, zeros_hbm,
             out_hbm, src_r, dst_r, rows_v, acc_sh, gsem, isem, ssem) = refs
        c = lax.axis_index("c")
        s = lax.axis_index("s")
        w = s * NCORES + c
        if nchunk % NW == 0:
            n_w = nchunk // NW
        else:
            n_w = nchunk // NW + jnp.where(w < nchunk % NW, 1, 0)

        def start_idx_dyn(i, b):
            off = (w + i * NW) * k
            pltpu.async_copy(src_hbm.at[pl.ds(off, k)], src_r.at[b],
                             isem.at[b])
            pltpu.async_copy(dst_hbm.at[pl.ds(off, k)], dst_r.at[b],
                             isem.at[b])

        def wait_idx(b):
            pltpu.make_async_copy(src_hbm.at[pl.ds(0, k)], src_r.at[b],
                                  isem.at[b]).wait()
            pltpu.make_async_copy(dst_hbm.at[pl.ds(0, k)], dst_r.at[b],
                                  isem.at[b]).wait()

        def start_gather(b):
            pltpu.async_copy(y_hbm.at[src_r.at[b]], rows_v.at[b], gsem.at[b])

        def wait_gather(b):
            pltpu.make_async_copy(y_hbm.at[src_r.at[b]], rows_v.at[b],
                                  gsem.at[b]).wait()

        def start_scatter(b):
            pltpu.async_copy(rows_v.at[b], acc_sh.at[dst_r.at[b]],
                             ssem.at[b], add=True)
            if with_deg:
                pltpu.async_copy(ones_v, deg_sh.at[dst_r.at[b]],
                                 ssem.at[b], add=True)

        def wait_scatter(b):
            pltpu.make_async_copy(rows_v.at[b], acc_sh.at[dst_r.at[b]],
                                  ssem.at[b]).wait()
            if with_deg:
                pltpu.make_async_copy(ones_v, deg_sh.at[dst_r.at[b]],
                                      ssem.at[b]).wait()

        for j in range(nbuf):
            start_idx_dyn(j, j)
        r0 = s * ROWS_PER_TILE
        pltpu.sync_copy(zeros_hbm.at[pl.ds(r0, ROWS_PER_TILE)],
                        acc_sh.at[pl.ds(r0, ROWS_PER_TILE)])
        if with_deg:
            pltpu.sync_copy(zeros16_hbm.at[pl.ds(r0, ROWS_PER_TILE)],
                            deg_sh.at[pl.ds(r0, ROWS_PER_TILE)])
            pltpu.sync_copy(ones_hbm, ones_v)
        plsc.subcore_barrier()

        for j in range(nbuf - 1):
            wait_idx(j)
            start_gather(j)

        def chunk(g, carry):
            b = lax.rem(g, nbuf)
            bp = lax.rem(g + nbuf - 1, nbuf)
            bg = lax.rem(g + nbuf - 2, nbuf)

            @pl.when(g >= 1)
            def _retire():
                wait_scatter(bp)

            @pl.when((g >= 1) & (g + nbuf - 1 < n_w))
            def _issue_idx():
                start_idx_dyn(g + nbuf - 1, bp)

            @pl.when((g >= 1) & (g + nbuf - 2 < n_w))
            def _issue_gather():
                wait_idx(bg)
                start_gather(bg)

            wait_gather(b)
            start_scatter(b)
            return carry

        lax.fori_loop(0, n_w, chunk, 0)
        wait_scatter(lax.rem(n_w - 1, nbuf))
        plsc.subcore_barrier()

        pltpu.sync_copy(acc_sh.at[pl.ds(r0, ROWS_PER_TILE)],
                        out_hbm.at[c, pl.ds(r0, ROWS_PER_TILE)])
        if with_deg:
            pltpu.sync_copy(deg_sh.at[pl.ds(r0, ROWS_PER_TILE)],
                            deg_hbm.at[c, pl.ds(r0, ROWS_PER_TILE)])

    return pl.kernel(
        body, mesh=mesh, out_type=out_type, scratch_types=scratch,
        compiler_params=pltpu.CompilerParams(use_tc_tiling_on_sc=False))


BN = 1000


def _tc_pre(x, ws, wn, b):
    dm = x.shape[1]
    do = ws.shape[1]

    def body(x_ref, ws_ref, wn_ref, b_ref, z_ref, y_ref):
        xv = x_ref[...]
        z_ref[...] = jnp.dot(xv, ws_ref[...],
                             preferred_element_type=jnp.float32) + b_ref[...]
        y_ref[...] = jnp.dot(xv, wn_ref[...], preferred_element_type=jnp.float32)

    return pl.pallas_call(
        body,
        grid=(N // BN,),
        in_specs=[
            pl.BlockSpec((BN, dm), lambda i: (i, 0)),
            pl.BlockSpec((dm, do), lambda i: (0, 0)),
            pl.BlockSpec((dm, do), lambda i: (0, 0)),
            pl.BlockSpec((1, do), lambda i: (0, 0)),
        ],
        out_specs=[
            pl.BlockSpec((BN, do), lambda i: (i, 0)),
            pl.BlockSpec((BN, do), lambda i: (i, 0)),
        ],
        out_shape=[
            jax.ShapeDtypeStruct((N, do), jnp.float32),
            jax.ShapeDtypeStruct((N, do), jnp.float32),
        ],
    )(x, ws, wn, b.reshape(1, do))


def _combine_refs(z_ref, pa_ref, pb_ref, da_ref, db_ref):
    deg = da_ref[0][:, 0:1] + db_ref[0][:, 0:1]
    invd = 1.0 / jnp.maximum(deg, 1.0)
    return z_ref[...] + (pa_ref[0] + pb_ref[0]) * invd


def _tc_mid(z, p, dg, ws, wn, b):
    dm = z.shape[1]
    do = ws.shape[1]

    def body(z_ref, pa_ref, pb_ref, da_ref, db_ref, ws_ref, wn_ref, b_ref,
             z2_ref, y2_ref):
        h = jnp.maximum(_combine_refs(z_ref, pa_ref, pb_ref, da_ref, db_ref),
                        0.0)
        z2_ref[...] = jnp.dot(h, ws_ref[...],
                              preferred_element_type=jnp.float32) + b_ref[...]
        y2_ref[...] = jnp.dot(h, wn_ref[...], preferred_element_type=jnp.float32)

    return pl.pallas_call(
        body,
        grid=(N // BN,),
        in_specs=[
            pl.BlockSpec((BN, dm), lambda i: (i, 0)),
            pl.BlockSpec((1, BN, dm), lambda i: (0, i, 0)),
            pl.BlockSpec((1, BN, dm), lambda i: (1, i, 0)),
            pl.BlockSpec((1, BN, 16), lambda i: (0, i, 0)),
            pl.BlockSpec((1, BN, 16), lambda i: (1, i, 0)),
            pl.BlockSpec((dm, do), lambda i: (0, 0)),
            pl.BlockSpec((dm, do), lambda i: (0, 0)),
            pl.BlockSpec((1, do), lambda i: (0, 0)),
        ],
        out_specs=[
            pl.BlockSpec((BN, do), lambda i: (i, 0)),
            pl.BlockSpec((BN, do), lambda i: (i, 0)),
        ],
        out_shape=[
            jax.ShapeDtypeStruct((N, do), jnp.float32),
            jax.ShapeDtypeStruct((N, do), jnp.float32),
        ],
    )(z, p, p, dg, dg, ws, wn, b.reshape(1, do))


def _tc_final(z, p, dg, nc):
    dm = z.shape[1]

    def body(z_ref, pa_ref, pb_ref, da_ref, db_ref, o_ref):
        o_ref[...] = _combine_refs(z_ref, pa_ref, pb_ref, da_ref,
                                   db_ref)[:, :nc]

    return pl.pallas_call(
        body,
        grid=(N // BN,),
        in_specs=[
            pl.BlockSpec((BN, dm), lambda i: (i, 0)),
            pl.BlockSpec((1, BN, dm), lambda i: (0, i, 0)),
            pl.BlockSpec((1, BN, dm), lambda i: (1, i, 0)),
            pl.BlockSpec((1, BN, 16), lambda i: (0, i, 0)),
            pl.BlockSpec((1, BN, 16), lambda i: (1, i, 0)),
        ],
        out_specs=pl.BlockSpec((BN, nc), lambda i: (i, 0)),
        out_shape=jax.ShapeDtypeStruct((N, nc), jnp.float32),
    )(z, p, p, dg, dg)


def kernel(features, edge_index, Wself0, Wneigh0, b0, Wself1, Wneigh1, b1,
           Wself2, Wneigh2, b2):
    src = edge_index[0]
    dst = edge_index[1]
    zeros128 = jnp.zeros((NP, D), jnp.float32)
    zeros64 = jnp.zeros((NP, NC_PAD), jnp.float32)
    zeros16 = jnp.zeros((NP, 16), jnp.float32)
    ones16 = jnp.ones((80, 16), jnp.float32)

    ws2p = jnp.pad(Wself2, ((0, 0), (0, NC_PAD - Wself2.shape[1])))
    wn2p = jnp.pad(Wneigh2, ((0, 0), (0, NC_PAD - Wneigh2.shape[1])))
    b2p = jnp.pad(b2, (0, NC_PAD - b2.shape[0]))

    z0, y0 = _tc_pre(features, Wself0, Wneigh0, b0)
    p0, dg = _make_sc_agg(D, True, 80, 3)(y0, src, dst, zeros128,
                                          zeros16, ones16)
    z1, y1 = _tc_mid(z0, p0, dg, Wself1, Wneigh1, b1)
    (p1,) = _make_sc_agg(D, False, 80, 4)(y1, src, dst, zeros128)
    z2, y2 = _tc_mid(z1, p1, dg, ws2p, wn2p, b2p)
    (p2,) = _make_sc_agg(NC_PAD, False, 128, 8)(y2, src, dst, zeros64)
    return _tc_final(z2, p2, dg, Wself2.shape[1])


# speedup vs baseline: 1.0551x; 1.0551x over previous
import jax
import jax.numpy as jnp
from jax import lax
from jax.experimental import pallas as pl
from jax.experimental.pallas import tpu as pltpu
from jax.experimental.pallas import tpu_sc as plsc

N = 10000
NP = 10112
E = 320000
D = 128
NC_PAD = 64
NCORES = 2
NSUB = 16
NW = NCORES * NSUB
ROWS_PER_TILE = NP // NSUB


def _make_sc_agg(dv, with_deg, k, nbuf):
    mesh = plsc.VectorSubcoreMesh(core_axis_name="c", subcore_axis_name="s")

    nchunk = E // k

    out_type = [jax.ShapeDtypeStruct((NCORES, NP, dv), jnp.float32)]
    scratch = [
        pltpu.VMEM((nbuf, k), jnp.int32),
        pltpu.VMEM((nbuf, k), jnp.int32),
        pltpu.VMEM((nbuf, k, dv), jnp.float32),
        pltpu.VMEM_SHARED((NP, dv), jnp.float32),
        pltpu.SemaphoreType.DMA((nbuf,)),
        pltpu.SemaphoreType.DMA((nbuf,)),
        pltpu.SemaphoreType.DMA((nbuf,)),
    ]
    if with_deg:
        out_type.append(jax.ShapeDtypeStruct((NCORES, NP, 16), jnp.float32))
        scratch += [
            pltpu.VMEM((k, 16), jnp.float32),
            pltpu.VMEM_SHARED((NP, 16), jnp.float32),
        ]

    def body(*refs):
        if with_deg:
            (y_hbm, edge_hbm, zeros_hbm, zeros16_hbm, ones_hbm,
             out_hbm, deg_hbm, src_r, dst_r, rows_v, acc_sh, gsem, isem, ssem,
             ones_v, deg_sh) = refs
        else:
            (y_hbm, edge_hbm, zeros_hbm,
             out_hbm, src_r, dst_r, rows_v, acc_sh, gsem, isem, ssem) = refs
        c = lax.axis_index("c")
        s = lax.axis_index("s")
        w = s * NCORES + c
        if nchunk % NW == 0:
            n_w = nchunk // NW
        else:
            n_w = nchunk // NW + jnp.where(w < nchunk % NW, 1, 0)

        def start_idx_dyn(i, b):
            off = (w + i * NW) * k
            pltpu.async_copy(edge_hbm.at[0, pl.ds(off, k)], src_r.at[b],
                             isem.at[b])
            pltpu.async_copy(edge_hbm.at[1, pl.ds(off, k)], dst_r.at[b],
                             isem.at[b])

        def wait_idx(b):
            pltpu.make_async_copy(edge_hbm.at[0, pl.ds(0, k)], src_r.at[b],
                                  isem.at[b]).wait()
            pltpu.make_async_copy(edge_hbm.at[1, pl.ds(0, k)], dst_r.at[b],
                                  isem.at[b]).wait()

        def start_gather(b):
            pltpu.async_copy(y_hbm.at[src_r.at[b]], rows_v.at[b], gsem.at[b])

        def wait_gather(b):
            pltpu.make_async_copy(y_hbm.at[src_r.at[b]], rows_v.at[b],
                                  gsem.at[b]).wait()

        def start_scatter(b):
            pltpu.async_copy(rows_v.at[b], acc_sh.at[dst_r.at[b]],
                             ssem.at[b], add=True)
            if with_deg:
                pltpu.async_copy(ones_v, deg_sh.at[dst_r.at[b]],
                                 ssem.at[b], add=True)

        def wait_scatter(b):
            pltpu.make_async_copy(rows_v.at[b], acc_sh.at[dst_r.at[b]],
                                  ssem.at[b]).wait()
            if with_deg:
                pltpu.make_async_copy(ones_v, deg_sh.at[dst_r.at[b]],
                                      ssem.at[b]).wait()

        for j in range(nbuf):
            start_idx_dyn(j, j)
        r0 = s * ROWS_PER_TILE
        pltpu.sync_copy(zeros_hbm.at[pl.ds(r0, ROWS_PER_TILE)],
                        acc_sh.at[pl.ds(r0, ROWS_PER_TILE)])
        if with_deg:
            pltpu.sync_copy(zeros16_hbm.at[pl.ds(r0, ROWS_PER_TILE)],
                            deg_sh.at[pl.ds(r0, ROWS_PER_TILE)])
            pltpu.sync_copy(ones_hbm, ones_v)
        plsc.subcore_barrier()

        for j in range(nbuf - 1):
            wait_idx(j)
            start_gather(j)

        def chunk(g, carry):
            b = lax.rem(g, nbuf)
            bp = lax.rem(g + nbuf - 1, nbuf)
            bg = lax.rem(g + nbuf - 2, nbuf)

            @pl.when(g >= 1)
            def _retire():
                wait_scatter(bp)

            @pl.when((g >= 1) & (g + nbuf - 1 < n_w))
            def _issue_idx():
                start_idx_dyn(g + nbuf - 1, bp)

            @pl.when((g >= 1) & (g + nbuf - 2 < n_w))
            def _issue_gather():
                wait_idx(bg)
                start_gather(bg)

            wait_gather(b)
            start_scatter(b)
            return carry

        lax.fori_loop(0, n_w, chunk, 0)
        wait_scatter(lax.rem(n_w - 1, nbuf))
        plsc.subcore_barrier()

        pltpu.sync_copy(acc_sh.at[pl.ds(r0, ROWS_PER_TILE)],
                        out_hbm.at[c, pl.ds(r0, ROWS_PER_TILE)])
        if with_deg:
            pltpu.sync_copy(deg_sh.at[pl.ds(r0, ROWS_PER_TILE)],
                            deg_hbm.at[c, pl.ds(r0, ROWS_PER_TILE)])

    return pl.kernel(
        body, mesh=mesh, out_type=out_type, scratch_types=scratch,
        compiler_params=pltpu.CompilerParams(use_tc_tiling_on_sc=False))


BN = 2000


def _tc_pre(x, ws, wn, b):
    dm = x.shape[1]
    do = ws.shape[1]

    def body(x_ref, ws_ref, wn_ref, b_ref, z_ref, y_ref):
        xv = x_ref[...]
        z_ref[...] = jnp.dot(xv, ws_ref[...],
                             preferred_element_type=jnp.float32) + b_ref[...]
        y_ref[...] = jnp.dot(xv, wn_ref[...], preferred_element_type=jnp.float32)

    return pl.pallas_call(
        body,
        grid=(N // BN,),
        in_specs=[
            pl.BlockSpec((BN, dm), lambda i: (i, 0)),
            pl.BlockSpec((dm, do), lambda i: (0, 0)),
            pl.BlockSpec((dm, do), lambda i: (0, 0)),
            pl.BlockSpec((1, do), lambda i: (0, 0)),
        ],
        out_specs=[
            pl.BlockSpec((BN, do), lambda i: (i, 0)),
            pl.BlockSpec((BN, do), lambda i: (i, 0)),
        ],
        out_shape=[
            jax.ShapeDtypeStruct((N, do), jnp.float32),
            jax.ShapeDtypeStruct((N, do), jnp.float32),
        ],
    )(x, ws, wn, b.reshape(1, do))


def _combine_refs(z_ref, pa_ref, pb_ref, da_ref, db_ref):
    deg = da_ref[0][:, 0:1] + db_ref[0][:, 0:1]
    invd = 1.0 / jnp.maximum(deg, 1.0)
    return z_ref[...] + (pa_ref[0] + pb_ref[0]) * invd


def _tc_mid(z, p, dg, ws, wn, b):
    dm = z.shape[1]
    do = ws.shape[1]

    def body(z_ref, pa_ref, pb_ref, da_ref, db_ref, ws_ref, wn_ref, b_ref,
             z2_ref, y2_ref):
        h = jnp.maximum(_combine_refs(z_ref, pa_ref, pb_ref, da_ref, db_ref),
                        0.0)
        z2_ref[...] = jnp.dot(h, ws_ref[...],
                              preferred_element_type=jnp.float32) + b_ref[...]
        y2_ref[...] = jnp.dot(h, wn_ref[...], preferred_element_type=jnp.float32)

    return pl.pallas_call(
        body,
        grid=(N // BN,),
        in_specs=[
            pl.BlockSpec((BN, dm), lambda i: (i, 0)),
            pl.BlockSpec((1, BN, dm), lambda i: (0, i, 0)),
            pl.BlockSpec((1, BN, dm), lambda i: (1, i, 0)),
            pl.BlockSpec((1, BN, 16), lambda i: (0, i, 0)),
            pl.BlockSpec((1, BN, 16), lambda i: (1, i, 0)),
            pl.BlockSpec((dm, do), lambda i: (0, 0)),
            pl.BlockSpec((dm, do), lambda i: (0, 0)),
            pl.BlockSpec((1, do), lambda i: (0, 0)),
        ],
        out_specs=[
            pl.BlockSpec((BN, do), lambda i: (i, 0)),
            pl.BlockSpec((BN, do), lambda i: (i, 0)),
        ],
        out_shape=[
            jax.ShapeDtypeStruct((N, do), jnp.float32),
            jax.ShapeDtypeStruct((N, do), jnp.float32),
        ],
    )(z, p, p, dg, dg, ws, wn, b.reshape(1, do))


def _tc_final(z, p, dg, nc):
    dm = z.shape[1]

    def body(z_ref, pa_ref, pb_ref, da_ref, db_ref, o_ref):
        o_ref[...] = _combine_refs(z_ref, pa_ref, pb_ref, da_ref,
                                   db_ref)[:, :nc]

    return pl.pallas_call(
        body,
        grid=(N // BN,),
        in_specs=[
            pl.BlockSpec((BN, dm), lambda i: (i, 0)),
            pl.BlockSpec((1, BN, dm), lambda i: (0, i, 0)),
            pl.BlockSpec((1, BN, dm), lambda i: (1, i, 0)),
            pl.BlockSpec((1, BN, 16), lambda i: (0, i, 0)),
            pl.BlockSpec((1, BN, 16), lambda i: (1, i, 0)),
        ],
        out_specs=pl.BlockSpec((BN, nc), lambda i: (i, 0)),
        out_shape=jax.ShapeDtypeStruct((N, nc), jnp.float32),
    )(z, p, p, dg, dg)


def kernel(features, edge_index, Wself0, Wneigh0, b0, Wself1, Wneigh1, b1,
           Wself2, Wneigh2, b2):
    zeros128 = jnp.zeros((NP, D), jnp.float32)
    zeros64 = jnp.zeros((NP, NC_PAD), jnp.float32)
    zeros16 = jnp.zeros((NP, 16), jnp.float32)
    ones16 = jnp.ones((80, 16), jnp.float32)

    ws2p = jnp.pad(Wself2, ((0, 0), (0, NC_PAD - Wself2.shape[1])))
    wn2p = jnp.pad(Wneigh2, ((0, 0), (0, NC_PAD - Wneigh2.shape[1])))
    b2p = jnp.pad(b2, (0, NC_PAD - b2.shape[0]))

    z0, y0 = _tc_pre(features, Wself0, Wneigh0, b0)
    p0, dg = _make_sc_agg(D, True, 80, 3)(y0, edge_index, zeros128,
                                          zeros16, ones16)
    z1, y1 = _tc_mid(z0, p0, dg, Wself1, Wneigh1, b1)
    (p1,) = _make_sc_agg(D, False, 80, 4)(y1, edge_index, zeros128)
    z2, y2 = _tc_mid(z1, p1, dg, ws2p, wn2p, b2p)
    (p2,) = _make_sc_agg(NC_PAD, False, 128, 8)(y2, edge_index, zeros64)
    return _tc_final(z2, p2, dg, Wself2.shape[1])
